# initial kernel scaffold (unmeasured)
import jax
import jax.numpy as jnp
from jax import lax
from jax.experimental import pallas as pl
from jax.experimental.pallas import tpu as pltpu

N_DEV = 32
N_R = 16
N_L = 15


def kernel(x, w_mat):
    m_per, k = x.shape
    _, n_per = w_mat.shape

    def body(
        x_ref,
        w_ref,
        out_ref,
        buf_r,
        buf_l,
        send_sem_r,
        recv_sem_r,
        send_sem_l,
        recv_sem_l,
        credit_r,
        credit_l,
    ):
        my = lax.axis_index("i")
        right = lax.rem(my + 1, N_DEV)
        left = lax.rem(my + N_DEV - 1, N_DEV)

        barrier_sem = pltpu.get_barrier_semaphore()
        for nbr in (left, right):
            pl.semaphore_signal(
                barrier_sem,
                inc=1,
                device_id=(nbr,),
                device_id_type=pl.DeviceIdType.MESH,
            )
        pl.semaphore_wait(barrier_sem, 2)

        buf_r[0, :, :] = x_ref[:, :]
        buf_l[0, :, :] = x_ref[:, :]

        out_ref[pl.ds(my * m_per, m_per), :] = jnp.dot(
            x_ref[:, :], w_ref[:, :], preferred_element_type=jnp.float32
        )

        for h in range(N_R):
            s = h % 2
            d = (h + 1) % 2
            do_left = h < N_L

            if h >= 2:
                pl.semaphore_wait(credit_r, 1)
            rdma_r = pltpu.make_async_remote_copy(
                src_ref=buf_r.at[s],
                dst_ref=buf_r.at[d],
                send_sem=send_sem_r.at[s],
                recv_sem=recv_sem_r.at[d],
                device_id=(right,),
                device_id_type=pl.DeviceIdType.MESH,
            )
            rdma_r.start()

            if do_left:
                if h >= 2:
                    pl.semaphore_wait(credit_l, 1)
                rdma_l = pltpu.make_async_remote_copy(
                    src_ref=buf_l.at[s],
                    dst_ref=buf_l.at[d],
                    send_sem=send_sem_l.at[s],
                    recv_sem=recv_sem_l.at[d],
                    device_id=(left,),
                    device_id_type=pl.DeviceIdType.MESH,
                )
                rdma_l.start()

            rdma_r.wait()
            if do_left:
                rdma_l.wait()

            origin_r = lax.rem(my + N_DEV - (h + 1), N_DEV)
            out_ref[pl.ds(origin_r * m_per, m_per), :] = jnp.dot(
                buf_r[d, :, :], w_ref[:, :], preferred_element_type=jnp.float32
            )
            if h <= N_R - 3:
                pl.semaphore_signal(
                    credit_r,
                    inc=1,
                    device_id=(left,),
                    device_id_type=pl.DeviceIdType.MESH,
                )
            if do_left:
                origin_l = lax.rem(my + (h + 1), N_DEV)
                out_ref[pl.ds(origin_l * m_per, m_per), :] = jnp.dot(
                    buf_l[d, :, :], w_ref[:, :], preferred_element_type=jnp.float32
                )
                if h <= N_L - 3:
                    pl.semaphore_signal(
                        credit_l,
                        inc=1,
                        device_id=(right,),
                        device_id_type=pl.DeviceIdType.MESH,
                    )

    out_shape = jax.ShapeDtypeStruct((N_DEV * m_per, n_per), jnp.float32)
    return pl.pallas_call(
        body,
        out_shape=out_shape,
        in_specs=[
            pl.BlockSpec(memory_space=pltpu.VMEM),
            pl.BlockSpec(memory_space=pltpu.VMEM),
        ],
        out_specs=pl.BlockSpec(memory_space=pltpu.VMEM),
        scratch_shapes=[
            pltpu.VMEM((2, m_per, k), jnp.float32),
            pltpu.VMEM((2, m_per, k), jnp.float32),
            pltpu.SemaphoreType.DMA((2,)),
            pltpu.SemaphoreType.DMA((2,)),
            pltpu.SemaphoreType.DMA((2,)),
            pltpu.SemaphoreType.DMA((2,)),
            pltpu.SemaphoreType.REGULAR,
            pltpu.SemaphoreType.REGULAR,
        ],
        compiler_params=pltpu.CompilerParams(collective_id=0),
    )(x, w_mat)


# baseline (device time: 781635 ns/iter reference)
import jax
import jax.numpy as jnp
from jax import lax
from jax.experimental import pallas as pl
from jax.experimental.pallas import tpu as pltpu

N_DEV = 32
N_R = 16
N_L = 15


def kernel(x, w_mat):
    m_per, k = x.shape
    _, n_per = w_mat.shape

    def body(
        x_ref,
        w_ref,
        out_ref,
        buf_r,
        buf_l,
        send_sem_r,
        recv_sem_r,
        send_sem_l,
        recv_sem_l,
        credit_r,
        credit_l,
    ):
        my = lax.axis_index("i")
        right = lax.rem(my + 1, N_DEV)
        left = lax.rem(my + N_DEV - 1, N_DEV)

        barrier_sem = pltpu.get_barrier_semaphore()
        for nbr in (left, right):
            pl.semaphore_signal(
                barrier_sem,
                inc=1,
                device_id=(nbr,),
                device_id_type=pl.DeviceIdType.MESH,
            )
        pl.semaphore_wait(barrier_sem, 2)

        buf_r[0, :, :] = x_ref[:, :]
        buf_l[0, :, :] = x_ref[:, :]

        out_ref[pl.ds(my * m_per, m_per), :] = jnp.dot(
            x_ref[:, :], w_ref[:, :], preferred_element_type=jnp.float32
        )

        for h in range(N_R):
            s = h % 2
            d = (h + 1) % 2
            do_left = h < N_L

            if h >= 1:
                pl.semaphore_wait(credit_r, 1)
            rdma_r = pltpu.make_async_remote_copy(
                src_ref=buf_r.at[s],
                dst_ref=buf_r.at[d],
                send_sem=send_sem_r.at[s],
                recv_sem=recv_sem_r.at[d],
                device_id=(right,),
                device_id_type=pl.DeviceIdType.MESH,
            )
            rdma_r.start()

            if do_left:
                if h >= 1:
                    pl.semaphore_wait(credit_l, 1)
                rdma_l = pltpu.make_async_remote_copy(
                    src_ref=buf_l.at[s],
                    dst_ref=buf_l.at[d],
                    send_sem=send_sem_l.at[s],
                    recv_sem=recv_sem_l.at[d],
                    device_id=(left,),
                    device_id_type=pl.DeviceIdType.MESH,
                )
                rdma_l.start()

            rdma_r.wait()
            if do_left:
                rdma_l.wait()

            if h <= N_R - 2:
                pl.semaphore_signal(
                    credit_r,
                    inc=1,
                    device_id=(left,),
                    device_id_type=pl.DeviceIdType.MESH,
                )
            if do_left and h <= N_L - 2:
                pl.semaphore_signal(
                    credit_l,
                    inc=1,
                    device_id=(right,),
                    device_id_type=pl.DeviceIdType.MESH,
                )

            origin_r = lax.rem(my + N_DEV - (h + 1), N_DEV)
            out_ref[pl.ds(origin_r * m_per, m_per), :] = jnp.dot(
                buf_r[d, :, :], w_ref[:, :], preferred_element_type=jnp.float32
            )
            if do_left:
                origin_l = lax.rem(my + (h + 1), N_DEV)
                out_ref[pl.ds(origin_l * m_per, m_per), :] = jnp.dot(
                    buf_l[d, :, :], w_ref[:, :], preferred_element_type=jnp.float32
                )

    out_shape = jax.ShapeDtypeStruct((N_DEV * m_per, n_per), jnp.float32)
    return pl.pallas_call(
        body,
        out_shape=out_shape,
        in_specs=[
            pl.BlockSpec(memory_space=pltpu.VMEM),
            pl.BlockSpec(memory_space=pltpu.VMEM),
        ],
        out_specs=pl.BlockSpec(memory_space=pltpu.VMEM),
        scratch_shapes=[
            pltpu.VMEM((2, m_per, k), jnp.float32),
            pltpu.VMEM((2, m_per, k), jnp.float32),
            pltpu.SemaphoreType.DMA((2,)),
            pltpu.SemaphoreType.DMA((2,)),
            pltpu.SemaphoreType.DMA((2,)),
            pltpu.SemaphoreType.DMA((2,)),
            pltpu.SemaphoreType.REGULAR,
            pltpu.SemaphoreType.REGULAR,
        ],
        compiler_params=pltpu.CompilerParams(collective_id=0),
    )(x, w_mat)
